# baseline (speedup 1.0000x reference)
import math

import jax
import jax.numpy as jnp
from jax.experimental import pallas as pl
from jax.experimental.pallas import tpu as pltpu


def _round_up(x, m):
    return (x + m - 1) // m * m


def _pad2(a, rows, cols):
    if a.shape == (rows, cols):
        return a
    return jnp.pad(a, ((0, rows - a.shape[0]), (0, cols - a.shape[1])))


def _fold_wf(w, theta, nhidden, h_pad):
    wf = theta * w + (1.0 - theta) * jnp.eye(nhidden, dtype=jnp.float32)
    return _pad2(wf, h_pad, h_pad).astype(jnp.bfloat16)


def _gcnii_kernel(x_ref, w0_ref, b0_ref, adj_ref, wf_ref, w1_ref, b1_ref,
                  o_ref, a8_ref, h8_ref, h0a_ref, inv_ref,
                  *, alpha, n_slabs, slab, nlayers):
    i = pl.program_id(0)

    @pl.when(i == 0)
    def _():
        h0 = jnp.maximum(
            jnp.dot(x_ref[...], w0_ref[...], preferred_element_type=jnp.float32)
            + b0_ref[...], 0.0)
        h0a_ref[...] = (alpha * h0).astype(jnp.bfloat16)
        h8_ref[0] = jnp.minimum(h0, 448.0).astype(h8_ref.dtype)

    @pl.when(i < n_slabs)
    def _():
        rows = pl.ds(i * slab, slab)
        a_f32 = adj_ref[...]
        a8 = a_f32.astype(a8_ref.dtype)
        a8_ref[rows, :] = a8
        deg = jnp.sum(a_f32, axis=1, keepdims=True)
        inv = (1.0 - alpha) / jnp.maximum(deg, 0.5)
        inv_ref[rows, :] = inv

        hi = jnp.dot(a8, h8_ref[0], preferred_element_type=jnp.float32)
        u = inv * hi + h0a_ref[rows, :].astype(jnp.float32)
        h_new = jnp.maximum(
            jnp.dot(u.astype(jnp.bfloat16), wf_ref[...],
                    preferred_element_type=jnp.float32), 0.0)
        h8_ref[1, rows, :] = jnp.minimum(h_new, 448.0).astype(h8_ref.dtype)

    @pl.when(i >= n_slabs)
    def _():
        l = i - (n_slabs - 1)
        p = jax.lax.rem(l, 2)
        hi = jnp.dot(a8_ref[...], h8_ref[p],
                     preferred_element_type=jnp.float32)
        u = inv_ref[...] * hi + h0a_ref[...].astype(jnp.float32)
        h_new = jnp.maximum(
            jnp.dot(u.astype(jnp.bfloat16), wf_ref[...],
                    preferred_element_type=jnp.float32), 0.0)
        h8_ref[1 - p] = jnp.minimum(h_new, 448.0).astype(h8_ref.dtype)

        @pl.when(l == nlayers - 1)
        def _():
            y = (jnp.dot(h_new.astype(jnp.bfloat16), w1_ref[...],
                         preferred_element_type=jnp.float32) + b1_ref[...])
            o_ref[...] = y


def kernel(w_fc0, b_fc0, w_fc1, b_fc1, conv_w_0, conv_w_1, conv_w_2, conv_w_3,
           conv_w_4, conv_w_5, conv_w_6, conv_w_7, x, adj, g):
    del g
    lamda, alpha = 0.5, 0.1
    conv_ws = [conv_w_0, conv_w_1, conv_w_2, conv_w_3,
               conv_w_4, conv_w_5, conv_w_6, conv_w_7]
    n, nfeat = x.shape
    nhidden = w_fc0.shape[1]
    nclass = w_fc1.shape[1]
    nlayers = len(conv_ws)

    n_pad = _round_up(n, 1024)
    f_pad = _round_up(nfeat, 128)
    h_pad = _round_up(nhidden, 128)
    c_pad = _round_up(nclass, 128)
    slab = 256
    n_slabs = n_pad // slab
    n_steps = n_slabs + (nlayers - 1)

    x_bf = _pad2(x, n_pad, f_pad).astype(jnp.bfloat16)
    adj_p = _pad2(adj, n_pad, n_pad)
    w0_bf = _pad2(w_fc0, f_pad, h_pad).astype(jnp.bfloat16)
    b0 = _pad2(b_fc0, 1, h_pad)
    w1_bf = _pad2(w_fc1, h_pad, c_pad).astype(jnp.bfloat16)
    b1 = _pad2(b_fc1, 1, c_pad)
    wf_stack = jnp.stack([
        _fold_wf(w, math.log(lamda / (i + 1) + 1.0), nhidden, h_pad)
        for i, w in enumerate(conv_ws)], axis=0)

    def _adj_idx(i):
        return (jnp.minimum(i, n_slabs - 1), 0)

    def _wf_idx(i):
        return (jnp.where(i < n_slabs, 0, i - (n_slabs - 1)), 0, 0)

    body = lambda *refs: _gcnii_kernel(
        *refs, alpha=alpha, n_slabs=n_slabs, slab=slab, nlayers=nlayers)
    out = pl.pallas_call(
        body,
        out_shape=jax.ShapeDtypeStruct((n_pad, c_pad), jnp.float32),
        grid=(n_steps,),
        in_specs=[
            pl.BlockSpec((n_pad, f_pad), lambda i: (0, 0)),
            pl.BlockSpec((f_pad, h_pad), lambda i: (0, 0)),
            pl.BlockSpec((1, h_pad), lambda i: (0, 0)),
            pl.BlockSpec((slab, n_pad), _adj_idx),
            pl.BlockSpec((None, h_pad, h_pad), _wf_idx),
            pl.BlockSpec((h_pad, c_pad), lambda i: (0, 0)),
            pl.BlockSpec((1, c_pad), lambda i: (0, 0)),
        ],
        out_specs=pl.BlockSpec((n_pad, c_pad), lambda i: (0, 0)),
        scratch_shapes=[pltpu.VMEM((n_pad, n_pad), jnp.float8_e4m3fn),
                        pltpu.VMEM((2, n_pad, h_pad), jnp.float8_e4m3fn),
                        pltpu.VMEM((n_pad, h_pad), jnp.bfloat16),
                        pltpu.VMEM((n_pad, 1), jnp.float32)],
        compiler_params=pltpu.CompilerParams(
            dimension_semantics=("arbitrary",),
            vmem_limit_bytes=67043328),
    )(x_bf, w0_bf, b0, adj_p, wf_stack, w1_bf, b1)
    return out[:n, :nclass]


# speedup vs baseline: 1.0247x; 1.0247x over previous
import math

import jax
import jax.numpy as jnp
from jax.experimental import pallas as pl
from jax.experimental.pallas import tpu as pltpu


def _round_up(x, m):
    return (x + m - 1) // m * m


def _pad2(a, rows, cols):
    if a.shape == (rows, cols):
        return a
    return jnp.pad(a, ((0, rows - a.shape[0]), (0, cols - a.shape[1])))


def _fold_wf(w, theta, nhidden, h_pad):
    wf = theta * w + (1.0 - theta) * jnp.eye(nhidden, dtype=jnp.float32)
    return _pad2(wf, h_pad, h_pad).astype(jnp.bfloat16)


def _gcnii_kernel(x_ref, w0_ref, b0_ref, adj_ref, wf_ref, w1_ref, b1_ref,
                  o_ref, a8_ref, h8_ref, h0a_ref, inv_ref,
                  *, alpha, n_slabs, slab, nlayers):
    i = pl.program_id(0)

    @pl.when(i == 0)
    def _():
        h0 = jnp.maximum(
            jnp.dot(x_ref[...], w0_ref[...], preferred_element_type=jnp.float32)
            + b0_ref[...], 0.0)
        h0a_ref[...] = (alpha * h0).astype(jnp.bfloat16)
        h8_ref[0] = jnp.minimum(h0, 448.0).astype(h8_ref.dtype)

    @pl.when(i < n_slabs)
    def _():
        rows = pl.ds(i * slab, slab)
        a_f32 = adj_ref[...]
        a8 = a_f32.astype(a8_ref.dtype)
        a8_ref[rows, :] = a8
        deg = jnp.sum(a_f32, axis=1, keepdims=True)
        inv = (1.0 - alpha) / jnp.maximum(deg, 0.5)
        inv_ref[rows, :] = inv

        hi = jnp.dot(a8, h8_ref[0], preferred_element_type=jnp.float32)
        u = inv * hi + h0a_ref[rows, :].astype(jnp.float32)
        h_new = jnp.maximum(
            jnp.dot(u.astype(jnp.bfloat16), wf_ref[...],
                    preferred_element_type=jnp.float32), 0.0)
        h8_ref[1, rows, :] = jnp.minimum(h_new, 448.0).astype(h8_ref.dtype)

    def _layer(l, src, dst):
        n_pad = a8_ref.shape[0]
        cs = n_pad // 4
        rhs = src[...]
        for s in range(4):
            rows = pl.ds(s * cs, cs)
            hi = jnp.dot(a8_ref[rows, :], rhs,
                         preferred_element_type=jnp.float32)
            u = inv_ref[rows, :] * hi + h0a_ref[rows, :].astype(jnp.float32)
            h_new = jnp.maximum(
                jnp.dot(u.astype(jnp.bfloat16), wf_ref[...],
                        preferred_element_type=jnp.float32), 0.0)
            dst[rows, :] = jnp.minimum(h_new, 448.0).astype(h8_ref.dtype)

            @pl.when(l == nlayers - 1)
            def _():
                y = (jnp.dot(h_new.astype(jnp.bfloat16), w1_ref[...],
                             preferred_element_type=jnp.float32) + b1_ref[...])
                o_ref[rows, :] = y

    @pl.when(i >= n_slabs)
    def _():
        l = i - (n_slabs - 1)
        p = jax.lax.rem(l, 2)

        @pl.when(p == 1)
        def _():
            _layer(l, h8_ref.at[1], h8_ref.at[0])

        @pl.when(p == 0)
        def _():
            _layer(l, h8_ref.at[0], h8_ref.at[1])


def kernel(w_fc0, b_fc0, w_fc1, b_fc1, conv_w_0, conv_w_1, conv_w_2, conv_w_3,
           conv_w_4, conv_w_5, conv_w_6, conv_w_7, x, adj, g):
    del g
    lamda, alpha = 0.5, 0.1
    conv_ws = [conv_w_0, conv_w_1, conv_w_2, conv_w_3,
               conv_w_4, conv_w_5, conv_w_6, conv_w_7]
    n, nfeat = x.shape
    nhidden = w_fc0.shape[1]
    nclass = w_fc1.shape[1]
    nlayers = len(conv_ws)

    n_pad = _round_up(n, 1024)
    f_pad = _round_up(nfeat, 128)
    h_pad = _round_up(nhidden, 128)
    c_pad = _round_up(nclass, 128)
    slab = 256
    n_slabs = n_pad // slab
    n_steps = n_slabs + (nlayers - 1)

    x_bf = _pad2(x, n_pad, f_pad).astype(jnp.bfloat16)
    adj_p = _pad2(adj, n_pad, n_pad)
    w0_bf = _pad2(w_fc0, f_pad, h_pad).astype(jnp.bfloat16)
    b0 = _pad2(b_fc0, 1, h_pad)
    w1_bf = _pad2(w_fc1, h_pad, c_pad).astype(jnp.bfloat16)
    b1 = _pad2(b_fc1, 1, c_pad)
    wf_stack = jnp.stack([
        _fold_wf(w, math.log(lamda / (i + 1) + 1.0), nhidden, h_pad)
        for i, w in enumerate(conv_ws)], axis=0)

    def _adj_idx(i):
        return (jnp.minimum(i, n_slabs - 1), 0)

    def _wf_idx(i):
        return (jnp.where(i < n_slabs, 0, i - (n_slabs - 1)), 0, 0)

    body = lambda *refs: _gcnii_kernel(
        *refs, alpha=alpha, n_slabs=n_slabs, slab=slab, nlayers=nlayers)
    out = pl.pallas_call(
        body,
        out_shape=jax.ShapeDtypeStruct((n_pad, c_pad), jnp.float32),
        grid=(n_steps,),
        in_specs=[
            pl.BlockSpec((n_pad, f_pad), lambda i: (0, 0)),
            pl.BlockSpec((f_pad, h_pad), lambda i: (0, 0)),
            pl.BlockSpec((1, h_pad), lambda i: (0, 0)),
            pl.BlockSpec((slab, n_pad), _adj_idx),
            pl.BlockSpec((None, h_pad, h_pad), _wf_idx),
            pl.BlockSpec((h_pad, c_pad), lambda i: (0, 0)),
            pl.BlockSpec((1, c_pad), lambda i: (0, 0)),
        ],
        out_specs=pl.BlockSpec((n_pad, c_pad), lambda i: (0, 0)),
        scratch_shapes=[pltpu.VMEM((n_pad, n_pad), jnp.float8_e4m3fn),
                        pltpu.VMEM((2, n_pad, h_pad), jnp.float8_e4m3fn),
                        pltpu.VMEM((n_pad, h_pad), jnp.bfloat16),
                        pltpu.VMEM((n_pad, 1), jnp.float32)],
        compiler_params=pltpu.CompilerParams(
            dimension_semantics=("arbitrary",),
            vmem_limit_bytes=67043328),
    )(x_bf, w0_bf, b0, adj_p, wf_stack, w1_bf, b1)
    return out[:n, :nclass]


# speedup vs baseline: 1.0431x; 1.0179x over previous
import math

import jax
import jax.numpy as jnp
from jax.experimental import pallas as pl
from jax.experimental.pallas import tpu as pltpu


def _round_up(x, m):
    return (x + m - 1) // m * m


def _pad2(a, rows, cols):
    if a.shape == (rows, cols):
        return a
    return jnp.pad(a, ((0, rows - a.shape[0]), (0, cols - a.shape[1])))


def _fold_wf(w, theta, nhidden, h_pad):
    wf = theta * w + (1.0 - theta) * jnp.eye(nhidden, dtype=jnp.float32)
    return _pad2(wf, h_pad, h_pad).astype(jnp.bfloat16)


def _gcnii_kernel(x_ref, w0_ref, b0_ref, adj_ref, wf_ref, w1_ref, b1_ref,
                  o_ref, a8_ref, h8_ref, h0a_ref, inv_ref,
                  *, alpha, n_slabs, slab, nlayers):
    i = pl.program_id(0)

    @pl.when(i == 0)
    def _():
        h0 = jnp.maximum(
            jnp.dot(x_ref[...], w0_ref[...], preferred_element_type=jnp.float32)
            + b0_ref[...], 0.0)
        h0a_ref[...] = (alpha * h0).astype(jnp.bfloat16)
        h8_ref[0] = jnp.minimum(h0, 448.0).astype(h8_ref.dtype)

    @pl.when(i < n_slabs)
    def _():
        rhs = h8_ref[0]
        for s in range(2):
            sub = slab // 2
            rows = pl.ds(i * slab + s * sub, sub)
            a_f32 = adj_ref[pl.ds(s * sub, sub), :]
            a8 = a_f32.astype(a8_ref.dtype)
            a8_ref[rows, :] = a8
            deg = jnp.sum(a_f32, axis=1, keepdims=True)
            inv = (1.0 - alpha) / jnp.maximum(deg, 0.5)
            inv_ref[rows, :] = inv

            hi = jnp.dot(a8, rhs, preferred_element_type=jnp.float32)
            u = inv * hi + h0a_ref[rows, :].astype(jnp.float32)
            h_new = jnp.maximum(
                jnp.dot(u.astype(jnp.bfloat16), wf_ref[...],
                        preferred_element_type=jnp.float32), 0.0)
            h8_ref[1, rows, :] = jnp.minimum(h_new, 448.0).astype(h8_ref.dtype)

    def _layer(l, src, dst):
        n_pad = a8_ref.shape[0]
        cs = n_pad // 4
        rhs = src[...]
        for s in range(4):
            rows = pl.ds(s * cs, cs)
            hi = jnp.dot(a8_ref[rows, :], rhs,
                         preferred_element_type=jnp.float32)
            u = inv_ref[rows, :] * hi + h0a_ref[rows, :].astype(jnp.float32)
            h_new = jnp.maximum(
                jnp.dot(u.astype(jnp.bfloat16), wf_ref[...],
                        preferred_element_type=jnp.float32), 0.0)
            dst[rows, :] = jnp.minimum(h_new, 448.0).astype(h8_ref.dtype)

            @pl.when(l == nlayers - 1)
            def _():
                y = (jnp.dot(h_new.astype(jnp.bfloat16), w1_ref[...],
                             preferred_element_type=jnp.float32) + b1_ref[...])
                o_ref[rows, :] = y

    @pl.when(i >= n_slabs)
    def _():
        l = i - (n_slabs - 1)
        p = jax.lax.rem(l, 2)

        @pl.when(p == 1)
        def _():
            _layer(l, h8_ref.at[1], h8_ref.at[0])

        @pl.when(p == 0)
        def _():
            _layer(l, h8_ref.at[0], h8_ref.at[1])


def kernel(w_fc0, b_fc0, w_fc1, b_fc1, conv_w_0, conv_w_1, conv_w_2, conv_w_3,
           conv_w_4, conv_w_5, conv_w_6, conv_w_7, x, adj, g):
    del g
    lamda, alpha = 0.5, 0.1
    conv_ws = [conv_w_0, conv_w_1, conv_w_2, conv_w_3,
               conv_w_4, conv_w_5, conv_w_6, conv_w_7]
    n, nfeat = x.shape
    nhidden = w_fc0.shape[1]
    nclass = w_fc1.shape[1]
    nlayers = len(conv_ws)

    n_pad = _round_up(n, 1024)
    f_pad = _round_up(nfeat, 128)
    h_pad = _round_up(nhidden, 128)
    c_pad = _round_up(nclass, 128)
    slab = 512
    n_slabs = n_pad // slab
    n_steps = n_slabs + (nlayers - 1)

    x_bf = _pad2(x, n_pad, f_pad).astype(jnp.bfloat16)
    adj_p = _pad2(adj, n_pad, n_pad)
    w0_bf = _pad2(w_fc0, f_pad, h_pad).astype(jnp.bfloat16)
    b0 = _pad2(b_fc0, 1, h_pad)
    w1_bf = _pad2(w_fc1, h_pad, c_pad).astype(jnp.bfloat16)
    b1 = _pad2(b_fc1, 1, c_pad)
    wf_stack = jnp.stack([
        _fold_wf(w, math.log(lamda / (i + 1) + 1.0), nhidden, h_pad)
        for i, w in enumerate(conv_ws)], axis=0)

    def _adj_idx(i):
        return (jnp.minimum(i, n_slabs - 1), 0)

    def _wf_idx(i):
        return (jnp.where(i < n_slabs, 0, i - (n_slabs - 1)), 0, 0)

    body = lambda *refs: _gcnii_kernel(
        *refs, alpha=alpha, n_slabs=n_slabs, slab=slab, nlayers=nlayers)
    out = pl.pallas_call(
        body,
        out_shape=jax.ShapeDtypeStruct((n_pad, c_pad), jnp.float32),
        grid=(n_steps,),
        in_specs=[
            pl.BlockSpec((n_pad, f_pad), lambda i: (0, 0)),
            pl.BlockSpec((f_pad, h_pad), lambda i: (0, 0)),
            pl.BlockSpec((1, h_pad), lambda i: (0, 0)),
            pl.BlockSpec((slab, n_pad), _adj_idx),
            pl.BlockSpec((None, h_pad, h_pad), _wf_idx),
            pl.BlockSpec((h_pad, c_pad), lambda i: (0, 0)),
            pl.BlockSpec((1, c_pad), lambda i: (0, 0)),
        ],
        out_specs=pl.BlockSpec((n_pad, c_pad), lambda i: (0, 0)),
        scratch_shapes=[pltpu.VMEM((n_pad, n_pad), jnp.float8_e4m3fn),
                        pltpu.VMEM((2, n_pad, h_pad), jnp.float8_e4m3fn),
                        pltpu.VMEM((n_pad, h_pad), jnp.bfloat16),
                        pltpu.VMEM((n_pad, 1), jnp.float32)],
        compiler_params=pltpu.CompilerParams(
            dimension_semantics=("arbitrary",),
            vmem_limit_bytes=67043328),
    )(x_bf, w0_bf, b0, adj_p, wf_stack, w1_bf, b1)
    return out[:n, :nclass]


# speedup vs baseline: 1.0483x; 1.0050x over previous
import math

import jax
import jax.numpy as jnp
from jax.experimental import pallas as pl
from jax.experimental.pallas import tpu as pltpu


def _round_up(x, m):
    return (x + m - 1) // m * m


def _pad2(a, rows, cols):
    if a.shape == (rows, cols):
        return a
    return jnp.pad(a, ((0, rows - a.shape[0]), (0, cols - a.shape[1])))


def _fold_wf(w, theta, nhidden, h_pad):
    wf = theta * w + (1.0 - theta) * jnp.eye(nhidden, dtype=jnp.float32)
    return _pad2(wf, h_pad, h_pad).astype(jnp.bfloat16)


def _gcnii_kernel(x_ref, w0_ref, b0_ref, adj_a_ref, adj_b_ref, wf_ref,
                  w1_ref, b1_ref, o_ref, a8_ref, h8_ref, h0a_ref, inv_ref,
                  *, alpha, n_stream, slab, nlayers):
    i = pl.program_id(0)
    n_pad = a8_ref.shape[0]

    @pl.when(i == 0)
    def _():
        h0 = jnp.maximum(
            jnp.dot(x_ref[...], w0_ref[...], preferred_element_type=jnp.float32)
            + b0_ref[...], 0.0)
        h0a_ref[...] = (alpha * h0).astype(jnp.bfloat16)
        h8_ref[0] = jnp.minimum(h0, 448.0).astype(h8_ref.dtype)

    @pl.when(i < n_stream)
    def _():
        rhs = h8_ref[0]
        for a_ref, base in ((adj_a_ref, 0), (adj_b_ref, n_pad // 2)):
            rows = pl.ds(base + i * slab, slab)
            a_f32 = a_ref[...]
            a8 = a_f32.astype(a8_ref.dtype)
            a8_ref[rows, :] = a8
            deg = jnp.sum(a_f32, axis=1, keepdims=True)
            inv = (1.0 - alpha) / jnp.maximum(deg, 0.5)
            inv_ref[rows, :] = inv

            hi = jnp.dot(a8, rhs, preferred_element_type=jnp.float32)
            u = inv * hi + h0a_ref[rows, :].astype(jnp.float32)
            h_new = jnp.maximum(
                jnp.dot(u.astype(jnp.bfloat16), wf_ref[...],
                        preferred_element_type=jnp.float32), 0.0)
            h8_ref[1, rows, :] = jnp.minimum(h_new, 448.0).astype(h8_ref.dtype)

    def _layer(l, src, dst):
        cs = n_pad // 4
        rhs = src[...]
        for s in range(4):
            rows = pl.ds(s * cs, cs)
            hi = jnp.dot(a8_ref[rows, :], rhs,
                         preferred_element_type=jnp.float32)
            u = inv_ref[rows, :] * hi + h0a_ref[rows, :].astype(jnp.float32)
            h_new = jnp.maximum(
                jnp.dot(u.astype(jnp.bfloat16), wf_ref[...],
                        preferred_element_type=jnp.float32), 0.0)
            dst[rows, :] = jnp.minimum(h_new, 448.0).astype(h8_ref.dtype)

            @pl.when(l == nlayers - 1)
            def _():
                y = (jnp.dot(h_new.astype(jnp.bfloat16), w1_ref[...],
                             preferred_element_type=jnp.float32) + b1_ref[...])
                o_ref[rows, :] = y

    @pl.when(i >= n_stream)
    def _():
        l = i - (n_stream - 1)
        p = jax.lax.rem(l, 2)

        @pl.when(p == 1)
        def _():
            _layer(l, h8_ref.at[1], h8_ref.at[0])

        @pl.when(p == 0)
        def _():
            _layer(l, h8_ref.at[0], h8_ref.at[1])


def kernel(w_fc0, b_fc0, w_fc1, b_fc1, conv_w_0, conv_w_1, conv_w_2, conv_w_3,
           conv_w_4, conv_w_5, conv_w_6, conv_w_7, x, adj, g):
    del g
    lamda, alpha = 0.5, 0.1
    conv_ws = [conv_w_0, conv_w_1, conv_w_2, conv_w_3,
               conv_w_4, conv_w_5, conv_w_6, conv_w_7]
    n, nfeat = x.shape
    nhidden = w_fc0.shape[1]
    nclass = w_fc1.shape[1]
    nlayers = len(conv_ws)

    n_pad = _round_up(n, 1024)
    f_pad = _round_up(nfeat, 128)
    h_pad = _round_up(nhidden, 128)
    c_pad = _round_up(nclass, 128)
    slab = 256
    n_stream = (n_pad // 2) // slab
    n_steps = n_stream + (nlayers - 1)

    x_bf = _pad2(x, n_pad, f_pad).astype(jnp.bfloat16)
    adj_p = _pad2(adj, n_pad, n_pad)
    w0_bf = _pad2(w_fc0, f_pad, h_pad).astype(jnp.bfloat16)
    b0 = _pad2(b_fc0, 1, h_pad)
    w1_bf = _pad2(w_fc1, h_pad, c_pad).astype(jnp.bfloat16)
    b1 = _pad2(b_fc1, 1, c_pad)
    wf_stack = jnp.stack([
        _fold_wf(w, math.log(lamda / (i + 1) + 1.0), nhidden, h_pad)
        for i, w in enumerate(conv_ws)], axis=0)

    def _adj_a_idx(i):
        return (jnp.minimum(i, n_stream - 1), 0)

    def _adj_b_idx(i):
        return (n_stream + jnp.minimum(i, n_stream - 1), 0)

    def _wf_idx(i):
        return (jnp.where(i < n_stream, 0, i - (n_stream - 1)), 0, 0)

    body = lambda *refs: _gcnii_kernel(
        *refs, alpha=alpha, n_stream=n_stream, slab=slab, nlayers=nlayers)
    out = pl.pallas_call(
        body,
        out_shape=jax.ShapeDtypeStruct((n_pad, c_pad), jnp.float32),
        grid=(n_steps,),
        in_specs=[
            pl.BlockSpec((n_pad, f_pad), lambda i: (0, 0)),
            pl.BlockSpec((f_pad, h_pad), lambda i: (0, 0)),
            pl.BlockSpec((1, h_pad), lambda i: (0, 0)),
            pl.BlockSpec((slab, n_pad), _adj_a_idx),
            pl.BlockSpec((slab, n_pad), _adj_b_idx),
            pl.BlockSpec((None, h_pad, h_pad), _wf_idx),
            pl.BlockSpec((h_pad, c_pad), lambda i: (0, 0)),
            pl.BlockSpec((1, c_pad), lambda i: (0, 0)),
        ],
        out_specs=pl.BlockSpec((n_pad, c_pad), lambda i: (0, 0)),
        scratch_shapes=[pltpu.VMEM((n_pad, n_pad), jnp.float8_e4m3fn),
                        pltpu.VMEM((2, n_pad, h_pad), jnp.float8_e4m3fn),
                        pltpu.VMEM((n_pad, h_pad), jnp.bfloat16),
                        pltpu.VMEM((n_pad, 1), jnp.float32)],
        compiler_params=pltpu.CompilerParams(
            dimension_semantics=("arbitrary",),
            vmem_limit_bytes=67043328),
    )(x_bf, w0_bf, b0, adj_p, adj_p, wf_stack, w1_bf, b1)
    return out[:n, :nclass]
